# Initial kernel scaffold; baseline (speedup 1.0000x reference)
import functools

import jax
import jax.numpy as jnp
import numpy as np
from jax import lax
from jax.experimental import pallas as pl
from jax.experimental.pallas import tpu as pltpu
from jax.experimental.pallas import tpu_sc as plsc

_N = 10000
_D = 128
_E = 320000
_BN_EPS = 1e-5
_BN_SCALE = float(1.0 / np.sqrt(1.0 + _BN_EPS))

_NC = 2
_NS = 16
_NW = _NC * _NS
_EPT = _E // _NW
_CH = 128
_NFULL = _EPT // _CH
_TAIL = _EPT - _NFULL * _CH
_RPS = _N // _NS
_ZR = 125


def _sc_segment_sum(h, src, dst):
    mesh = plsc.VectorSubcoreMesh(
        core_axis_name="c", subcore_axis_name="s",
        num_cores=_NC, num_subcores=_NS)

    @functools.partial(
        pl.kernel,
        out_type=jax.ShapeDtypeStruct((_NC, _N, _D), jnp.float32),
        mesh=mesh,
        scratch_types=[
            pltpu.VMEM_SHARED((_N, _D), jnp.float32),
            pltpu.VMEM((_CH,), jnp.int32),
            pltpu.VMEM((_CH,), jnp.int32),
            pltpu.VMEM((_CH, _D), jnp.float32),
            pltpu.VMEM((_TAIL,), jnp.int32),
            pltpu.VMEM((_TAIL,), jnp.int32),
            pltpu.VMEM((_TAIL, _D), jnp.float32),
            pltpu.VMEM((_ZR, _D), jnp.float32),
        ],
    )
    def k(h_hbm, src_hbm, dst_hbm, out_hbm,
          agg_sh, src_v, dst_v, rows_v, srct_v, dstt_v, rowst_v, zero_v):
        c = lax.axis_index("c")
        s = lax.axis_index("s")
        wid = s * _NC + c

        @pl.loop(0, _ZR)
        def _(r):
            @pl.loop(0, _D, step=16)
            def _(c0):
                zero_v[r, pl.ds(c0, 16)] = jnp.zeros((16,), jnp.float32)

        rbase = s * _RPS

        @pl.loop(0, _RPS // _ZR)
        def _(kk):
            pltpu.sync_copy(zero_v, agg_sh.at[pl.ds(rbase + kk * _ZR, _ZR)])

        plsc.subcore_barrier()

        ebase = wid * _EPT

        @pl.loop(0, _NFULL)
        def _(j):
            off = ebase + j * _CH
            pltpu.sync_copy(src_hbm.at[pl.ds(off, _CH)], src_v)
            pltpu.sync_copy(dst_hbm.at[pl.ds(off, _CH)], dst_v)
            pltpu.sync_copy(h_hbm.at[src_v], rows_v)
            pltpu.sync_copy(rows_v, agg_sh.at[dst_v], add=True)

        offt = ebase + _NFULL * _CH
        pltpu.sync_copy(src_hbm.at[pl.ds(offt, _TAIL)], srct_v)
        pltpu.sync_copy(dst_hbm.at[pl.ds(offt, _TAIL)], dstt_v)
        pltpu.sync_copy(h_hbm.at[srct_v], rowst_v)
        pltpu.sync_copy(rowst_v, agg_sh.at[dstt_v], add=True)

        plsc.subcore_barrier()

        pltpu.sync_copy(agg_sh.at[pl.ds(rbase, _RPS)],
                        out_hbm.at[c, pl.ds(rbase, _RPS)])

    return k(h, src, dst)


def _mlp(h, agg, W1, b1r, W2f, b2f, epsv, relu_out):
    BR = 1000

    def body(eps_ref, h_ref, agg_ref, w1_ref, b1_ref, w2_ref, b2_ref, out_ref):
        z = eps_ref[...] * h_ref[...] + agg_ref[0] + agg_ref[1]
        z = jnp.dot(z, w1_ref[...], preferred_element_type=jnp.float32) + b1_ref[...]
        z = jnp.maximum(z, 0.0)
        z = jnp.dot(z, w2_ref[...], preferred_element_type=jnp.float32) + b2_ref[...]
        if relu_out:
            z = jnp.maximum(z, 0.0)
        out_ref[...] = z

    return pl.pallas_call(
        body,
        grid=(_N // BR,),
        in_specs=[
            pl.BlockSpec((1, _D), lambda i: (0, 0)),
            pl.BlockSpec((BR, _D), lambda i: (i, 0)),
            pl.BlockSpec((_NC, BR, _D), lambda i: (0, i, 0)),
            pl.BlockSpec((_D, _D), lambda i: (0, 0)),
            pl.BlockSpec((1, _D), lambda i: (0, 0)),
            pl.BlockSpec((_D, _D), lambda i: (0, 0)),
            pl.BlockSpec((1, _D), lambda i: (0, 0)),
        ],
        out_specs=pl.BlockSpec((BR, _D), lambda i: (i, 0)),
        out_shape=jax.ShapeDtypeStruct((_N, _D), jnp.float32),
    )(epsv, h, agg, W1, b1r, W2f, b2f)


def kernel(x, edge_index,
           W1_0, b1_0, W2_0, b2_0, eps_0, gamma_0, beta_0,
           W1_1, b1_1, W2_1, b2_1, eps_1, gamma_1, beta_1,
           W1_2, b1_2, W2_2, b2_2, eps_2, gamma_2, beta_2):
    src = edge_index[0]
    dst = edge_index[1]
    layers = [
        (W1_0, b1_0, W2_0, b2_0, eps_0, gamma_0, beta_0),
        (W1_1, b1_1, W2_1, b2_1, eps_1, gamma_1, beta_1),
        (W1_2, b1_2, W2_2, b2_2, eps_2, gamma_2, beta_2),
    ]
    h = x
    for i, (W1, b1, W2, b2, eps, gamma, beta) in enumerate(layers):
        agg = _sc_segment_sum(h, src, dst)
        gs = gamma * _BN_SCALE
        W2f = W2 * gs[None, :]
        b2f = (b2 * gs + beta).reshape(1, _D)
        epsv = jnp.broadcast_to(1.0 + eps, (1, _D)).astype(jnp.float32)
        h = _mlp(h, agg, W1, b1.reshape(1, _D), W2f, b2f, epsv, i < 2)
    return h


# speedup vs baseline: 6.3302x; 6.3302x over previous
import functools

import jax
import jax.numpy as jnp
import numpy as np
from jax import lax
from jax.experimental import pallas as pl
from jax.experimental.pallas import tpu as pltpu
from jax.experimental.pallas import tpu_sc as plsc

_N = 10000
_D = 128
_E = 320000
_BN_EPS = 1e-5
_BN_SCALE = float(1.0 / np.sqrt(1.0 + _BN_EPS))

_NC = 2
_NS = 16
_NW = _NC * _NS
_EPT = _E // _NW
_CH = 128
_NFULL = _EPT // _CH
_TAIL = _EPT - _NFULL * _CH
_NPAD = 10240
_RPS = _NPAD // _NS
_RLAST = _N - (_NS - 1) * _RPS
_ZR = 128


def _sc_segment_sum(h, src, dst):
    mesh = plsc.VectorSubcoreMesh(
        core_axis_name="c", subcore_axis_name="s",
        num_cores=_NC, num_subcores=_NS)

    @functools.partial(
        pl.kernel,
        out_type=jax.ShapeDtypeStruct((_NC, _N, _D), jnp.float32),
        mesh=mesh,
        scratch_types=[
            pltpu.VMEM_SHARED((_NPAD, _D), jnp.float32),
            pltpu.VMEM((_CH,), jnp.int32),
            pltpu.VMEM((_CH,), jnp.int32),
            pltpu.VMEM((_CH, _D), jnp.float32),
            pltpu.VMEM((_TAIL,), jnp.int32),
            pltpu.VMEM((_TAIL,), jnp.int32),
            pltpu.VMEM((_TAIL, _D), jnp.float32),
            pltpu.VMEM((_ZR, _D), jnp.float32),
        ],
    )
    def k(h_hbm, src_hbm, dst_hbm, out_hbm,
          agg_sh, src_v, dst_v, rows_v, srct_v, dstt_v, rowst_v, zero_v):
        c = lax.axis_index("c")
        s = lax.axis_index("s")
        wid = s * _NC + c

        @pl.loop(0, _ZR)
        def _(r):
            @pl.loop(0, _D, step=16)
            def _(c0):
                zero_v[r, pl.ds(c0, 16)] = jnp.zeros((16,), jnp.float32)

        rbase = s * _RPS

        @pl.loop(0, _RPS // _ZR)
        def _(kk):
            pltpu.sync_copy(zero_v, agg_sh.at[pl.ds(rbase + kk * _ZR, _ZR)])

        plsc.subcore_barrier()

        ebase = wid * _EPT

        @pl.loop(0, _NFULL)
        def _(j):
            off = ebase + j * _CH
            pltpu.sync_copy(src_hbm.at[pl.ds(off, _CH)], src_v)
            pltpu.sync_copy(dst_hbm.at[pl.ds(off, _CH)], dst_v)
            pltpu.sync_copy(h_hbm.at[src_v], rows_v)
            pltpu.sync_copy(rows_v, agg_sh.at[dst_v], add=True)

        offt = ebase + _NFULL * _CH
        pltpu.sync_copy(src_hbm.at[pl.ds(offt, _TAIL)], srct_v)
        pltpu.sync_copy(dst_hbm.at[pl.ds(offt, _TAIL)], dstt_v)
        pltpu.sync_copy(h_hbm.at[srct_v], rowst_v)
        pltpu.sync_copy(rowst_v, agg_sh.at[dstt_v], add=True)

        plsc.subcore_barrier()

        @pl.when(s < _NS - 1)
        def _():
            pltpu.sync_copy(agg_sh.at[pl.ds(rbase, _RPS)],
                            out_hbm.at[c, pl.ds(rbase, _RPS)])

        @pl.when(s == _NS - 1)
        def _():
            pltpu.sync_copy(agg_sh.at[pl.ds(rbase, _RLAST)],
                            out_hbm.at[c, pl.ds(rbase, _RLAST)])

    return k(h, src, dst)


def _mlp(h, agg, W1, b1r, W2f, b2f, epsv, relu_out):
    BR = 1000

    def body(eps_ref, h_ref, agg_ref, w1_ref, b1_ref, w2_ref, b2_ref, out_ref):
        z = eps_ref[...] * h_ref[...] + agg_ref[0] + agg_ref[1]
        z = jnp.dot(z, w1_ref[...], preferred_element_type=jnp.float32) + b1_ref[...]
        z = jnp.maximum(z, 0.0)
        z = jnp.dot(z, w2_ref[...], preferred_element_type=jnp.float32) + b2_ref[...]
        if relu_out:
            z = jnp.maximum(z, 0.0)
        out_ref[...] = z

    return pl.pallas_call(
        body,
        grid=(_N // BR,),
        in_specs=[
            pl.BlockSpec((1, _D), lambda i: (0, 0)),
            pl.BlockSpec((BR, _D), lambda i: (i, 0)),
            pl.BlockSpec((_NC, BR, _D), lambda i: (0, i, 0)),
            pl.BlockSpec((_D, _D), lambda i: (0, 0)),
            pl.BlockSpec((1, _D), lambda i: (0, 0)),
            pl.BlockSpec((_D, _D), lambda i: (0, 0)),
            pl.BlockSpec((1, _D), lambda i: (0, 0)),
        ],
        out_specs=pl.BlockSpec((BR, _D), lambda i: (i, 0)),
        out_shape=jax.ShapeDtypeStruct((_N, _D), jnp.float32),
    )(epsv, h, agg, W1, b1r, W2f, b2f)


def kernel(x, edge_index,
           W1_0, b1_0, W2_0, b2_0, eps_0, gamma_0, beta_0,
           W1_1, b1_1, W2_1, b2_1, eps_1, gamma_1, beta_1,
           W1_2, b1_2, W2_2, b2_2, eps_2, gamma_2, beta_2):
    src = edge_index[0]
    dst = edge_index[1]
    layers = [
        (W1_0, b1_0, W2_0, b2_0, eps_0, gamma_0, beta_0),
        (W1_1, b1_1, W2_1, b2_1, eps_1, gamma_1, beta_1),
        (W1_2, b1_2, W2_2, b2_2, eps_2, gamma_2, beta_2),
    ]
    h = x
    for i, (W1, b1, W2, b2, eps, gamma, beta) in enumerate(layers):
        agg = _sc_segment_sum(h, src, dst)
        gs = gamma * _BN_SCALE
        W2f = W2 * gs[None, :]
        b2f = (b2 * gs + beta).reshape(1, _D)
        epsv = jnp.broadcast_to(1.0 + eps, (1, _D)).astype(jnp.float32)
        h = _mlp(h, agg, W1, b1.reshape(1, _D), W2f, b2f, epsv, i < 2)
    return h
